# baseline (speedup 1.0000x reference)
import functools

import jax
import jax.numpy as jnp
from jax import lax
from jax.experimental import pallas as pl
from jax.experimental.pallas import tpu as pltpu
from jax.experimental.pallas import tpu_sc as plsc

NC = 2
NS = 16
NW = NC * NS
LANES = 16
K = 4
NG = 3
UNROLL = 8


def _make_kernel(B, S, V, D):
    SW = S // NW
    CK = SW // K
    VPR = D // LANES
    JBLK = VPR // UNROLL
    RPB = S // K

    mesh = plsc.VectorSubcoreMesh(core_axis_name="c", subcore_axis_name="s")

    scratch = (
        [pltpu.VMEM((B * CK, K), jnp.int32)]
        + [pltpu.VMEM((K, D), jnp.float32) for _ in range(NG * B)]
        + [pltpu.VMEM((K, D), jnp.float32) for _ in range(2)]
        + [pltpu.SemaphoreType.DMA for _ in range(2 * NG * B + 2)]
    )

    @functools.partial(
        pl.kernel,
        mesh=mesh,
        out_type=jax.ShapeDtypeStruct((B * S, D), jnp.float32),
        scratch_types=scratch,
    )
    def k(ids_hbm, word_hbm, pos_hbm, out_hbm, idx_all, *rest):
        o = rest[:NG * B]
        pbuf = rest[NG * B:NG * B + 2]
        gsem = rest[NG * B + 2:2 * NG * B + 2]
        wsem = rest[2 * NG * B + 2:3 * NG * B + 2]
        psem = rest[3 * NG * B + 2:3 * NG * B + 4]

        wid = lax.axis_index("s") * NC + lax.axis_index("c")
        seq_base = wid * SW

        for b in range(B):
            pltpu.async_copy(
                ids_hbm.at[pl.ds(b * RPB + wid * CK, CK)],
                idx_all.at[pl.ds(b * CK, CK)],
                psem[0],
            )
        for b in range(B):
            pltpu.make_async_copy(
                ids_hbm.at[pl.ds(b * RPB + wid * CK, CK)],
                idx_all.at[pl.ds(b * CK, CK)],
                psem[0],
            ).wait()

        def issue_gather(c, b, g):
            s = B * g + b
            pltpu.async_copy(
                word_hbm.at[idx_all.at[b * CK + c]], o[s], gsem[s]
            )

        def wait_gather(c, b, g):
            s = B * g + b
            pltpu.make_async_copy(
                word_hbm.at[idx_all.at[b * CK + c]], o[s], gsem[s]
            ).wait()

        def drain_writes(b, g):
            s = B * g + b
            pltpu.make_async_copy(
                o[s], out_hbm.at[pl.ds(seq_base, K)], wsem[s]
            ).wait()

        def issue_pos(c, p):
            pltpu.async_copy(
                pos_hbm.at[pl.ds(seq_base + c * K, K)], pbuf[p], psem[p]
            )

        def wait_pos(p):
            pltpu.make_async_copy(
                pos_hbm.at[pl.ds(seq_base, K)], pbuf[p], psem[p]
            ).wait()

        def fused_add(c, g, p):
            slots = [o[B * g + b] for b in range(B)]
            pb = pbuf[p]

            def row_body(r, _):
                def col_body(j, _):
                    base = j * (LANES * UNROLL)
                    xs = [
                        pb[r, pl.ds(base + u * LANES, LANES)]
                        for u in range(UNROLL)
                    ]
                    for u in range(UNROLL):
                        off = base + u * LANES
                        for ov in slots:
                            plsc.addupdate(ov.at[r, pl.ds(off, LANES)], xs[u])
                    return 0
                lax.fori_loop(0, JBLK, col_body, 0)
                for b in range(B):
                    s = B * g + b
                    pltpu.async_copy(
                        o[s].at[pl.ds(r, 1)],
                        out_hbm.at[pl.ds(b * S + seq_base + c * K + r, 1)],
                        wsem[s],
                    )
                return 0
            lax.fori_loop(0, K, row_body, 0)

        issue_pos(0, 0)
        issue_pos(1, 1)
        for b in range(B):
            issue_gather(0, b, 0)
        for b in range(B):
            issue_gather(1, b, 1)

        wait_pos(0)
        for b in range(B):
            wait_gather(0, b, 0)
        fused_add(0, 0, 0)
        issue_pos(2, 0)

        wait_pos(1)
        for b in range(B):
            wait_gather(1, b, 1)
        for b in range(B):
            issue_gather(2, b, 2)
        fused_add(1, 1, 1)
        issue_pos(3, 1)

        def do_chunk(c, g, p):
            wait_pos(p)
            for b in range(B):
                wait_gather(c, b, g)
            @pl.when(c + 1 < CK)
            def _():
                gn = (g + 1) % NG
                for b in range(B):
                    drain_writes(b, gn)
                    issue_gather(c + 1, b, gn)
            fused_add(c, g, p)

            @pl.when(c + 2 < CK)
            def _():
                issue_pos(c + 2, p)

        def step_body(t, _):
            c0 = 2 + 2 * NG * t
            for j in range(2 * NG):
                do_chunk(c0 + j, (2 + j) % NG, j % 2)
            return 0

        lax.fori_loop(0, (CK - 2) // (2 * NG), step_body, 0)

        for s in range(NG * B):
            pltpu.make_async_copy(
                o[s], out_hbm.at[pl.ds(seq_base, K)], wsem[s]
            ).wait()

    return k


def kernel(input_ids, word_table, pos_table):
    B, S = input_ids.shape
    V, D = word_table.shape
    ids2 = input_ids.reshape((B * S) // K, K).astype(jnp.int32)
    k = _make_kernel(B, S, V, D)
    out = k(ids2, word_table, pos_table)
    return out.reshape(B, S, D)


# speedup vs baseline: 1.0265x; 1.0265x over previous
import functools

import jax
import jax.numpy as jnp
from jax import lax
from jax.experimental import pallas as pl
from jax.experimental.pallas import tpu as pltpu
from jax.experimental.pallas import tpu_sc as plsc

NC = 2
NS = 16
NW = NC * NS
LANES = 16
K = 4
UNROLL = 8


def _make_kernel(B, S, V, D):
    SW = S // NW
    CK = SW // K
    VPR = D // LANES
    JBLK = VPR // UNROLL
    RPB = S // K

    mesh = plsc.VectorSubcoreMesh(core_axis_name="c", subcore_axis_name="s")

    scratch = (
        [pltpu.VMEM((B * CK, K), jnp.int32)]
        + [pltpu.VMEM((K, D), jnp.float32) for _ in range(2 * B)]
        + [pltpu.VMEM((K, D), jnp.float32) for _ in range(2)]
        + [pltpu.SemaphoreType.DMA for _ in range(4 * B + 2)]
    )

    @functools.partial(
        pl.kernel,
        mesh=mesh,
        out_type=jax.ShapeDtypeStruct((B * S, D), jnp.float32),
        scratch_types=scratch,
    )
    def k(ids_hbm, word_hbm, pos_hbm, out_hbm, idx_all, *rest):
        o = rest[:2 * B]
        pbuf = rest[2 * B:2 * B + 2]
        gsem = rest[2 * B + 2:4 * B + 2]
        wsem = rest[4 * B + 2:6 * B + 2]
        psem = rest[6 * B + 2:6 * B + 4]

        wid = lax.axis_index("s") * NC + lax.axis_index("c")
        seq_base = wid * SW

        for b in range(B):
            pltpu.async_copy(
                ids_hbm.at[pl.ds(b * RPB + wid * CK, CK)],
                idx_all.at[pl.ds(b * CK, CK)],
                psem[0],
            )
        for b in range(B):
            pltpu.make_async_copy(
                ids_hbm.at[pl.ds(b * RPB + wid * CK, CK)],
                idx_all.at[pl.ds(b * CK, CK)],
                psem[0],
            ).wait()

        def issue_gather(c, b, q):
            s = B * q + b
            pltpu.async_copy(
                word_hbm.at[idx_all.at[b * CK + c]], o[s], gsem[s]
            )

        def wait_gather(c, b, q):
            s = B * q + b
            pltpu.make_async_copy(
                word_hbm.at[idx_all.at[b * CK + c]], o[s], gsem[s]
            ).wait()

        def drain_writes(b, q):
            s = B * q + b
            pltpu.make_async_copy(
                o[s], out_hbm.at[pl.ds(seq_base, K)], wsem[s]
            ).wait()

        def issue_pos(c, q):
            pltpu.async_copy(
                pos_hbm.at[pl.ds(seq_base + c * K, K)], pbuf[q], psem[q]
            )

        def wait_pos(q):
            pltpu.make_async_copy(
                pos_hbm.at[pl.ds(seq_base, K)], pbuf[q], psem[q]
            ).wait()

        def fused_add(c, q):
            slots = [o[B * q + b] for b in range(B)]
            pb = pbuf[q]

            def row_body(r, _):
                def col_body(j, _):
                    base = j * (LANES * UNROLL)
                    xs = [
                        pb[r, pl.ds(base + u * LANES, LANES)]
                        for u in range(UNROLL)
                    ]
                    for u in range(UNROLL):
                        off = base + u * LANES
                        for ov in slots:
                            plsc.addupdate(ov.at[r, pl.ds(off, LANES)], xs[u])
                    return 0
                lax.fori_loop(0, JBLK, col_body, 0)
                for b in range(B):
                    s = B * q + b
                    pltpu.async_copy(
                        o[s].at[pl.ds(r, 1)],
                        out_hbm.at[pl.ds(b * S + seq_base + c * K + r, 1)],
                        wsem[s],
                    )
                return 0
            lax.fori_loop(0, K, row_body, 0)

        issue_pos(0, 0)
        issue_pos(1, 1)
        for b in range(B):
            issue_gather(0, b, 0)
        for b in range(B):
            issue_gather(1, b, 1)

        wait_pos(0)
        for b in range(B):
            wait_gather(0, b, 0)
        fused_add(0, 0)
        issue_pos(2, 0)

        wait_pos(1)
        for b in range(B):
            wait_gather(1, b, 1)
        for b in range(B):
            drain_writes(b, 0)
            issue_gather(2, b, 0)
        fused_add(1, 1)
        issue_pos(3, 1)

        def do_chunk(c, q):
            wait_pos(q)
            for b in range(B):
                wait_gather(c, b, q)
            @pl.when(c + 1 < CK)
            def _():
                for b in range(B):
                    drain_writes(b, 1 - q)
                    issue_gather(c + 1, b, 1 - q)
            fused_add(c, q)

            @pl.when(c + 2 < CK)
            def _():
                issue_pos(c + 2, q)

        def step_body(s2, _):
            do_chunk(2 * s2, 0)
            do_chunk(2 * s2 + 1, 1)
            return 0

        lax.fori_loop(1, CK // 2, step_body, 0)

        for s in range(2 * B):
            pltpu.make_async_copy(
                o[s], out_hbm.at[pl.ds(seq_base, K)], wsem[s]
            ).wait()

    return k


def kernel(input_ids, word_table, pos_table):
    B, S = input_ids.shape
    V, D = word_table.shape
    ids2 = input_ids.reshape((B * S) // K, K).astype(jnp.int32)
    k = _make_kernel(B, S, V, D)
    out = k(ids2, word_table, pos_table)
    return out.reshape(B, S, D)
